# baseline (speedup 1.0000x reference)
import functools

import jax
import jax.numpy as jnp
from jax import lax
from jax.experimental import pallas as pl
from jax.experimental.pallas import tpu as pltpu
from jax.experimental.pallas import tpu_sc as plsc

IN_F = 128
OUT_F = 64
BATCH = 1024
LANES = 16
NC = 2
NS = 16
NW = NC * NS
BPW = BATCH // NW
OG = OUT_F // LANES


def _tc_body(x_ref, lv_ref, bias_ref, c01_ref, c11_ref, dense_ref):
    lv = lv_ref[...]
    r1 = jnp.roll(lv, -1, axis=1)
    r2 = jnp.roll(lv, -2, axis=1)
    r3 = jnp.roll(lv, -3, axis=1)
    il = lax.broadcasted_iota(jnp.int32, (4 * IN_F, IN_F), 0)
    ic = lax.broadcasted_iota(jnp.int32, (4 * IN_F, IN_F), 1)
    sel = (il == 4 * ic).astype(jnp.float32)
    hi = jax.lax.Precision.HIGHEST
    c01_ref[...] = jnp.matmul(r2 - lv, sel, precision=hi)
    c11_ref[...] = jnp.matmul((lv - r1) + (r3 - r2), sel, precision=hi)
    c10 = jnp.matmul(r1 - lv, sel, precision=hi)
    l0c = jnp.matmul(lv, sel, precision=hi)
    bias2 = bias_ref[...] + jnp.sum(l0c, axis=1)[None, :]
    dense_ref[...] = jnp.matmul(x_ref[...], c10.T, precision=hi) + bias2


_tc_call = pl.pallas_call(
    _tc_body,
    out_shape=(
        jax.ShapeDtypeStruct((OUT_F, IN_F), jnp.float32),
        jax.ShapeDtypeStruct((OUT_F, IN_F), jnp.float32),
        jax.ShapeDtypeStruct((BATCH, OUT_F), jnp.float32),
    ),
)


_sc_mesh = plsc.VectorSubcoreMesh(core_axis_name="c", subcore_axis_name="s")


@functools.partial(
    pl.kernel,
    out_type=jax.ShapeDtypeStruct((BATCH, OUT_F), jnp.float32),
    mesh=_sc_mesh,
    compiler_params=pltpu.CompilerParams(needs_layout_passes=False),
    scratch_types=[
        pltpu.VMEM((BPW * IN_F,), jnp.float32),
        pltpu.VMEM((OUT_F, IN_F), jnp.float32),
        pltpu.VMEM((OUT_F, IN_F), jnp.float32),
        pltpu.VMEM((2 * OUT_F * IN_F,), jnp.int32),
        pltpu.VMEM((BPW, OUT_F), jnp.float32),
    ],
)
def _sc_kernel(x_hbm, c01_hbm, c11_hbm, m_hbm, dense_hbm, out_hbm,
               xv, cf01, cf11, mv, ov):
    wid = lax.axis_index("s") * NC + lax.axis_index("c")
    base = wid * BPW
    pltpu.sync_copy(x_hbm.at[pl.ds(base * IN_F, BPW * IN_F)], xv)
    pltpu.sync_copy(c01_hbm, cf01)
    pltpu.sync_copy(c11_hbm, cf11)
    pltpu.sync_copy(m_hbm, mv)
    pltpu.sync_copy(dense_hbm.at[pl.ds(base, BPW)], ov)

    lane = lax.broadcasted_iota(jnp.int32, (LANES,), 0)
    o_lane = [lane + og * LANES for og in range(OG)]
    om256 = [(lane + og * LANES) * (2 * IN_F) for og in range(OG)]
    bv0 = lane * IN_F
    bv1 = (lane + LANES) * IN_F

    def body_i(i, _):
        col = jnp.full((LANES,), i, jnp.int32)
        ms = jnp.full((LANES,), 2 * i + 1, jnp.int32)
        e0a = plsc.load_gather(xv, [bv0 + col])
        e0b = plsc.load_gather(xv, [bv1 + col])
        regs = []
        for og in range(OG):
            c01v = plsc.load_gather(cf01, [o_lane[og], col])
            c11v = plsc.load_gather(cf11, [o_lane[og], col])
            rv = plsc.load_gather(mv, [om256[og] + ms])
            regs.append((c01v, c11v, rv))

        @plsc.parallel_loop(0, BPW, unroll=8)
        def body_b(b):
            row = xv.at[pl.ds(b * IN_F, IN_F)]
            bidx = jnp.full((LANES,), jnp.bitwise_and(b, LANES - 1), jnp.int32)
            e0 = lax.gather(
                jnp.where(b < LANES, e0a, e0b), bidx[:, None],
                lax.GatherDimensionNumbers(
                    offset_dims=(), collapsed_slice_dims=(0,),
                    start_index_map=(0,)),
                slice_sizes=(1,),
                mode=lax.GatherScatterMode.PROMISE_IN_BOUNDS)
            for og in range(OG):
                c01v, c11v, rv = regs[og]
                e1 = plsc.load_gather(row, [rv])
                val = e1 * (c01v + c11v * e0)
                plsc.addupdate(ov.at[b, pl.ds(og * LANES, LANES)], val)

        return 0

    lax.fori_loop(0, IN_F, body_i, 0)
    pltpu.sync_copy(ov, out_hbm.at[pl.ds(base, BPW)])


def kernel(input, lut, bias, input_mask):
    lv = lut.reshape(OUT_F, 4 * IN_F)
    c01c, c11c, dense = _tc_call(input, lv, bias.reshape(1, OUT_F))
    return _sc_kernel(input.reshape(-1), c01c, c11c, input_mask, dense)


# speedup vs baseline: 1.0345x; 1.0345x over previous
import functools

import jax
import jax.numpy as jnp
from jax import lax
from jax.experimental import pallas as pl
from jax.experimental.pallas import tpu as pltpu
from jax.experimental.pallas import tpu_sc as plsc

IN_F = 128
OUT_F = 64
BATCH = 1024
LANES = 16
NC = 2
NS = 16
NW = NC * NS
BPW = BATCH // NW
OG = OUT_F // LANES


def _tc_body(x_ref, lv_ref, bias_ref, c01_ref, c11_ref, dense_ref):
    lv = lv_ref[...]
    r1 = jnp.roll(lv, -1, axis=1)
    r2 = jnp.roll(lv, -2, axis=1)
    r3 = jnp.roll(lv, -3, axis=1)
    il = lax.broadcasted_iota(jnp.int32, (4 * IN_F, IN_F), 0)
    ic = lax.broadcasted_iota(jnp.int32, (4 * IN_F, IN_F), 1)
    sel = (il == 4 * ic).astype(jnp.float32)
    hi = jax.lax.Precision.HIGHEST
    c01_ref[...] = jnp.matmul(r2 - lv, sel, precision=hi)
    c11_ref[...] = jnp.matmul((lv - r1) + (r3 - r2), sel, precision=hi)
    c10 = jnp.matmul(r1 - lv, sel, precision=hi)
    l0c = jnp.matmul(lv, sel, precision=hi)
    bias2 = bias_ref[...] + jnp.sum(l0c, axis=1)[None, :]
    dense_ref[...] = jnp.matmul(x_ref[...], c10.T, precision=hi) + bias2


_tc_call = pl.pallas_call(
    _tc_body,
    out_shape=(
        jax.ShapeDtypeStruct((OUT_F, IN_F), jnp.float32),
        jax.ShapeDtypeStruct((OUT_F, IN_F), jnp.float32),
        jax.ShapeDtypeStruct((BATCH, OUT_F), jnp.float32),
    ),
)


_sc_mesh = plsc.VectorSubcoreMesh(core_axis_name="c", subcore_axis_name="s")


@functools.partial(
    pl.kernel,
    out_type=jax.ShapeDtypeStruct((BATCH, OUT_F), jnp.float32),
    mesh=_sc_mesh,
    compiler_params=pltpu.CompilerParams(needs_layout_passes=False),
    scratch_types=[
        pltpu.VMEM((BPW * IN_F,), jnp.float32),
        pltpu.VMEM((OUT_F, IN_F), jnp.float32),
        pltpu.VMEM((OUT_F, IN_F), jnp.float32),
        pltpu.VMEM((2 * OUT_F * IN_F,), jnp.int32),
        pltpu.VMEM((BPW, OUT_F), jnp.float32),
    ],
)
def _sc_kernel(x_hbm, c01_hbm, c11_hbm, m_hbm, dense_hbm, out_hbm,
               xv, cf01, cf11, mv, ov):
    wid = lax.axis_index("s") * NC + lax.axis_index("c")
    base = wid * BPW
    pltpu.sync_copy(x_hbm.at[pl.ds(base * IN_F, BPW * IN_F)], xv)
    pltpu.sync_copy(c01_hbm, cf01)
    pltpu.sync_copy(c11_hbm, cf11)
    pltpu.sync_copy(m_hbm, mv)
    pltpu.sync_copy(dense_hbm.at[pl.ds(base, BPW)], ov)

    lane = lax.broadcasted_iota(jnp.int32, (LANES,), 0)
    o_lane = [lane + og * LANES for og in range(OG)]
    om256 = [(lane + og * LANES) * (2 * IN_F) for og in range(OG)]
    bv0 = lane * IN_F
    bv1 = (lane + LANES) * IN_F

    def body_i(i, _):
        col = jnp.full((LANES,), i, jnp.int32)
        ms = jnp.full((LANES,), 2 * i + 1, jnp.int32)
        e0a = plsc.load_gather(xv, [bv0 + col])
        e0b = plsc.load_gather(xv, [bv1 + col])
        regs = []
        for og in range(OG):
            c01v = plsc.load_gather(cf01, [o_lane[og], col])
            c11v = plsc.load_gather(cf11, [o_lane[og], col])
            rv = plsc.load_gather(mv, [om256[og] + ms])
            regs.append((c01v, c11v, rv))

        @plsc.parallel_loop(0, BPW, unroll=4)
        def body_b(b):
            row = xv.at[pl.ds(b * IN_F, IN_F)]
            bidx = jnp.full((LANES,), jnp.bitwise_and(b, LANES - 1), jnp.int32)
            e0 = lax.gather(
                jnp.where(b < LANES, e0a, e0b), bidx[:, None],
                lax.GatherDimensionNumbers(
                    offset_dims=(), collapsed_slice_dims=(0,),
                    start_index_map=(0,)),
                slice_sizes=(1,),
                mode=lax.GatherScatterMode.PROMISE_IN_BOUNDS)
            e1s = [plsc.load_gather(row, [regs[og][2]]) for og in range(OG)]
            for og in range(OG):
                c01v, c11v, _ = regs[og]
                val = e1s[og] * (c01v + c11v * e0)
                plsc.addupdate(ov.at[b, pl.ds(og * LANES, LANES)], val)

        return 0

    lax.fori_loop(0, IN_F, body_i, 0)
    pltpu.sync_copy(ov, out_hbm.at[pl.ds(base, BPW)])


def kernel(input, lut, bias, input_mask):
    lv = lut.reshape(OUT_F, 4 * IN_F)
    c01c, c11c, dense = _tc_call(input, lv, bias.reshape(1, OUT_F))
    return _sc_kernel(input.reshape(-1), c01c, c11c, input_mask, dense)


# speedup vs baseline: 1.0727x; 1.0369x over previous
import functools

import jax
import jax.numpy as jnp
from jax import lax
from jax.experimental import pallas as pl
from jax.experimental.pallas import tpu as pltpu
from jax.experimental.pallas import tpu_sc as plsc

IN_F = 128
OUT_F = 64
BATCH = 1024
LANES = 16
NC = 2
NS = 16
NW = NC * NS
BPW = BATCH // NW
OG = OUT_F // LANES


def _tc_body(x_ref, lv_ref, bias_ref, m_ref, pk_ref, dense_ref):
    lv = lv_ref[...]
    r1 = jnp.roll(lv, -1, axis=1)
    r2 = jnp.roll(lv, -2, axis=1)
    r3 = jnp.roll(lv, -3, axis=1)
    il = lax.broadcasted_iota(jnp.int32, (4 * IN_F, IN_F), 0)
    ic = lax.broadcasted_iota(jnp.int32, (4 * IN_F, IN_F), 1)
    sel = (il == 4 * ic).astype(jnp.float32)
    hi = jax.lax.Precision.HIGHEST
    pk_ref[0:OUT_F, :] = jnp.matmul(r2 - lv, sel, precision=hi)
    pk_ref[OUT_F:2 * OUT_F, :] = jnp.matmul(
        (lv - r1) + (r3 - r2), sel, precision=hi)
    pk_ref[2 * OUT_F:, :] = lax.bitcast_convert_type(
        m_ref[...], jnp.float32)
    c10 = jnp.matmul(r1 - lv, sel, precision=hi)
    l0c = jnp.matmul(lv, sel, precision=hi)
    bias2 = bias_ref[...] + jnp.sum(l0c, axis=1)[None, :]
    dense_ref[...] = jnp.matmul(x_ref[...], c10.T, precision=hi) + bias2


_tc_call = pl.pallas_call(
    _tc_body,
    out_shape=(
        jax.ShapeDtypeStruct((2 * OUT_F + IN_F, IN_F), jnp.float32),
        jax.ShapeDtypeStruct((BATCH, OUT_F), jnp.float32),
    ),
)


_sc_mesh = plsc.VectorSubcoreMesh(core_axis_name="c", subcore_axis_name="s")


@functools.partial(
    pl.kernel,
    out_type=jax.ShapeDtypeStruct((BATCH, OUT_F), jnp.float32),
    mesh=_sc_mesh,
    compiler_params=pltpu.CompilerParams(needs_layout_passes=False),
    scratch_types=[
        pltpu.VMEM((BPW * IN_F,), jnp.float32),
        pltpu.VMEM((2 * OUT_F + IN_F, IN_F), jnp.float32),
        pltpu.VMEM((BPW, OUT_F), jnp.float32),
    ],
)
def _sc_kernel(x_hbm, pk_hbm, dense_hbm, out_hbm, xv, pk, ov):
    wid = lax.axis_index("s") * NC + lax.axis_index("c")
    base = wid * BPW
    pltpu.sync_copy(x_hbm.at[pl.ds(base * IN_F, BPW * IN_F)], xv)
    pltpu.sync_copy(pk_hbm, pk)
    pltpu.sync_copy(dense_hbm.at[pl.ds(base, BPW)], ov)

    lane = lax.broadcasted_iota(jnp.int32, (LANES,), 0)
    o_lane = [lane + og * LANES for og in range(OG)]
    o_lane2 = [(lane + og * LANES) * 2 for og in range(OG)]
    bv0 = lane * IN_F
    bv1 = (lane + LANES) * IN_F

    def body_i(i, _):
        col = jnp.full((LANES,), i, jnp.int32)
        mrow = jnp.full((LANES,), 2 * OUT_F + (2 * i + 1) // IN_F, jnp.int32)
        mcol = jnp.full((LANES,), (2 * i + 1) % IN_F, jnp.int32)
        e0a = plsc.load_gather(xv, [bv0 + col])
        e0b = plsc.load_gather(xv, [bv1 + col])
        regs = []
        for og in range(OG):
            c01v = plsc.load_gather(pk, [o_lane[og], col])
            c11v = plsc.load_gather(pk, [o_lane[og] + OUT_F, col])
            rv = plsc.bitcast(
                plsc.load_gather(pk, [o_lane2[og] + mrow, mcol]), jnp.int32)
            regs.append((c01v, c11v, rv))

        @plsc.parallel_loop(0, BPW, unroll=4)
        def body_b(b):
            row = xv.at[pl.ds(b * IN_F, IN_F)]
            bidx = jnp.full((LANES,), jnp.bitwise_and(b, LANES - 1), jnp.int32)
            e0 = lax.gather(
                jnp.where(b < LANES, e0a, e0b), bidx[:, None],
                lax.GatherDimensionNumbers(
                    offset_dims=(), collapsed_slice_dims=(0,),
                    start_index_map=(0,)),
                slice_sizes=(1,),
                mode=lax.GatherScatterMode.PROMISE_IN_BOUNDS)
            for og in range(OG):
                c01v, c11v, rv = regs[og]
                e1 = plsc.load_gather(row, [rv])
                val = e1 * (c01v + c11v * e0)
                plsc.addupdate(ov.at[b, pl.ds(og * LANES, LANES)], val)

        return 0

    lax.fori_loop(0, IN_F, body_i, 0)
    pltpu.sync_copy(ov, out_hbm.at[pl.ds(base, BPW)])


def kernel(input, lut, bias, input_mask):
    lv = lut.reshape(OUT_F, 4 * IN_F)
    m2 = input_mask.reshape(IN_F, IN_F)
    packed, dense = _tc_call(input, lv, bias.reshape(1, OUT_F), m2)
    return _sc_kernel(input.reshape(-1), packed, dense)
